# baseline (speedup 1.0000x reference)
import functools

import jax
import jax.numpy as jnp
from jax import lax
from jax.experimental import pallas as pl
from jax.experimental.pallas import tpu as pltpu
from jax.experimental.pallas import tpu_sc as plsc

B, D, H, W = 4, 64, 64, 64
E = 32
DHW = D * H * W
N = B * DHW

NC, NS = 2, 16
NW = NC * NS
W_PER_B = NW // B
PER_W = DHW // W_PER_B
CHUNK = 1024
N_CHUNKS = PER_W // CHUNK
PITCH = CHUNK + 1

_mesh = plsc.VectorSubcoreMesh(
    core_axis_name="c", subcore_axis_name="s", num_cores=NC, num_subcores=NS
)


@functools.partial(
    pl.kernel,
    out_type=jax.ShapeDtypeStruct((B, E, DHW), jnp.float32),
    mesh=_mesh,
    scratch_types=[
        pltpu.VMEM((CHUNK,), jnp.int32),
        pltpu.VMEM((CHUNK,), jnp.int32),
        pltpu.VMEM((CHUNK, E), jnp.float32),
        pltpu.VMEM((CHUNK, E), jnp.float32),
        pltpu.VMEM((E, PITCH), jnp.float32),
        pltpu.SemaphoreType.DMA,
        pltpu.SemaphoreType.DMA,
        pltpu.SemaphoreType.DMA,
        pltpu.SemaphoreType.DMA,
        pltpu.SemaphoreType.DMA,
    ],
    compiler_params=pltpu.CompilerParams(
        use_tc_tiling_on_sc=False, needs_layout_passes=False
    ),
)
def _sc_fused(idx_hbm, table_hbm, out_hbm, idx_v0, idx_v1, rows_v0, rows_v1,
              trans_v, isem0, isem1, sem0, sem1, osem):
    wid = lax.axis_index("s") * NC + lax.axis_index("c")
    bb = wid // W_PER_B
    off = (wid % W_PER_B) * PER_W

    e_lo = lax.iota(jnp.int32, 16)
    e_hi = e_lo + 16

    def start_idx(k, idx_v, isem):
        pltpu.async_copy(
            idx_hbm.at[pl.ds(bb * DHW + off + k * CHUNK, CHUNK)], idx_v, isem)

    def start_gather(idx_v, isem, rows_v, sem):
        pltpu.make_async_copy(
            idx_hbm.at[pl.ds(0, CHUNK)], idx_v, isem).wait()
        pltpu.async_copy(table_hbm.at[idx_v], rows_v, sem)

    def wait_gather(idx_v, rows_v, sem):
        pltpu.make_async_copy(table_hbm.at[idx_v], rows_v, sem).wait()

    def transpose(rows_v):
        @functools.partial(plsc.parallel_loop, 0, CHUNK, unroll=16)
        def _transpose(j):
            jv = jnp.full((16,), j, jnp.int32)
            r0 = rows_v[j, pl.ds(0, 16)]
            r1 = rows_v[j, pl.ds(16, 16)]
            plsc.store_scatter(trans_v, [e_lo, jv], r0)
            plsc.store_scatter(trans_v, [e_hi, jv], r1)

    start_idx(0, idx_v0, isem0)
    start_idx(1, idx_v1, isem1)
    start_gather(idx_v0, isem0, rows_v0, sem0)

    @pl.loop(0, N_CHUNKS, step=2)
    def _pipeline(i):
        start_gather(idx_v1, isem1, rows_v1, sem1)
        wait_gather(idx_v0, rows_v0, sem0)

        @pl.when(i + 2 < N_CHUNKS)
        def _():
            start_idx(i + 2, idx_v0, isem0)

        @pl.when(i >= 1)
        def _():
            pltpu.make_async_copy(
                trans_v.at[:, pl.ds(0, CHUNK)],
                out_hbm.at[bb, :, pl.ds(off, CHUNK)], osem).wait()

        transpose(rows_v0)
        pltpu.async_copy(
            trans_v.at[:, pl.ds(0, CHUNK)],
            out_hbm.at[bb, :, pl.ds(off + i * CHUNK, CHUNK)], osem)

        @pl.when(i + 2 < N_CHUNKS)
        def _():
            start_gather(idx_v0, isem0, rows_v0, sem0)

        wait_gather(idx_v1, rows_v1, sem1)

        @pl.when(i + 3 < N_CHUNKS)
        def _():
            start_idx(i + 3, idx_v1, isem1)

        pltpu.make_async_copy(
            trans_v.at[:, pl.ds(0, CHUNK)],
            out_hbm.at[bb, :, pl.ds(off, CHUNK)], osem).wait()
        transpose(rows_v1)
        pltpu.async_copy(
            trans_v.at[:, pl.ds(0, CHUNK)],
            out_hbm.at[bb, :, pl.ds(off + (i + 1) * CHUNK, CHUNK)], osem)

    pltpu.make_async_copy(
        trans_v.at[:, pl.ds(0, CHUNK)],
        out_hbm.at[bb, :, pl.ds(off, CHUNK)], osem).wait()


def kernel(v, table):
    idx = v.reshape(N)
    out = _sc_fused(idx, table)
    return out.reshape(B, E, D, H, W)
